# Initial kernel scaffold; baseline (speedup 1.0000x reference)
import jax
import jax.numpy as jnp
from jax.experimental import pallas as pl


def kernel(x_nchw, block0_w, block0_b, block0_s, block0_t, block1_w, block1_b, block1_s, block1_t, block2_w, block2_b, block2_s, block2_t, block3_w, block3_b, block3_s, block3_t, fc1_w, fc1_b, fc2_w, fc2_b, fc3_w, fc3_b):
    raise NotImplementedError("write your pallas kernel here")


# speedup vs baseline: 7.9064x; 7.9064x over previous
import functools

import jax
import jax.numpy as jnp
from jax.experimental import pallas as pl
from jax.experimental.pallas import tpu as pltpu


def _conv_pool_kernel(x_ref, w_ref, b_ref, s_ref, t_ref, o_ref, *, H, Wh, C, OC):
    OH = H - 2
    PH = OH // 2
    PW = Wh - 1
    M2 = OH * Wh - 1

    X4 = jnp.concatenate(
        [x_ref[0, ki * Wh + s:ki * Wh + s + M2, :]
         for ki in range(3) for s in range(2)],
        axis=-1,
    )
    z = jnp.dot(X4, w_ref[...], preferred_element_type=jnp.float32)
    z = z + b_ref[...]
    z = jnp.maximum(z, 0.0)
    z = z * s_ref[...] + t_ref[...]
    zw = jnp.maximum(z[:, :OC], z[:, OC:]).astype(o_ref.dtype)
    zm = jnp.maximum(zw[:M2 - Wh, :], zw[Wh:, :])
    for ph in range(PH):
        o_ref[0, ph] = zm[2 * ph * Wh:2 * ph * Wh + PW, :]


def _expand_w_pairs(bw, bb, bs, bt, C, OC):
    w9 = bw.reshape(3, 3, C, OC)
    w4 = jnp.zeros((3, 4, C, 2, OC), bw.dtype)
    w4 = w4.at[:, 0:3, :, 0, :].set(w9)
    w4 = w4.at[:, 1:4, :, 1, :].set(w9)
    w4 = w4.reshape(12 * C, 2 * OC)
    pair = lambda v: jnp.concatenate([v, v], axis=1)
    return w4, pair(bb), pair(bs), pair(bt)


def _conv_block(x, bw, bb, bs, bt, C, OC):
    B, H, W, _ = x.shape
    Wh = W // 2
    PH = (H - 2) // 2
    PW = Wh - 1
    x2 = x.reshape(B, H * Wh, 2 * C)
    w4, b2, s2, t2 = _expand_w_pairs(bw, bb, bs, bt, C, OC)
    kern = functools.partial(_conv_pool_kernel, H=H, Wh=Wh, C=C, OC=OC)
    vec = pl.BlockSpec((1, 2 * OC), lambda b: (0, 0))
    return pl.pallas_call(
        kern,
        out_shape=jax.ShapeDtypeStruct((B, PH, PW, OC), jnp.bfloat16),
        grid=(B,),
        in_specs=[
            pl.BlockSpec((1, H * Wh, 2 * C), lambda b: (b, 0, 0)),
            pl.BlockSpec((12 * C, 2 * OC), lambda b: (0, 0)),
            vec, vec, vec,
        ],
        out_specs=pl.BlockSpec((1, PH, PW, OC), lambda b: (b, 0, 0, 0)),
        compiler_params=pltpu.CompilerParams(
            dimension_semantics=("parallel",)),
    )(x2, w4, b2, s2, t2)


def _fc_kernel(x_ref, w1_ref, b1_ref, w2_ref, b2_ref, w3_ref, b3_ref,
               o_ref, acc_ref, *, nk):
    k = pl.program_id(0)

    @pl.when(k == 0)
    def _():
        acc_ref[...] = jnp.zeros_like(acc_ref)

    acc_ref[...] += jnp.dot(x_ref[...], w1_ref[...],
                            preferred_element_type=jnp.float32)

    @pl.when(k == nk - 1)
    def _():
        h1 = jnp.maximum(acc_ref[...] + b1_ref[...], 0.0).astype(jnp.bfloat16)
        z2 = jnp.dot(h1, w2_ref[...], preferred_element_type=jnp.float32)
        h2 = jnp.maximum(z2 + b2_ref[...], 0.0).astype(jnp.bfloat16)
        z3 = jnp.dot(h2, w3_ref[...], preferred_element_type=jnp.float32)
        o_ref[...] = z3 + b3_ref[...]


def _fc_head(x, w1, b1, w2, b2, w3, b3, tk=4608):
    B, K = x.shape
    nk = K // tk
    N1 = w1.shape[1]
    N2 = w2.shape[1]
    N3 = w3.shape[1]
    return pl.pallas_call(
        functools.partial(_fc_kernel, nk=nk),
        out_shape=jax.ShapeDtypeStruct((B, N3), jnp.float32),
        grid=(nk,),
        in_specs=[
            pl.BlockSpec((B, tk), lambda k: (0, k)),
            pl.BlockSpec((tk, N1), lambda k: (k, 0)),
            pl.BlockSpec((1, N1), lambda k: (0, 0)),
            pl.BlockSpec((N1, N2), lambda k: (0, 0)),
            pl.BlockSpec((1, N2), lambda k: (0, 0)),
            pl.BlockSpec((N2, N3), lambda k: (0, 0)),
            pl.BlockSpec((1, N3), lambda k: (0, 0)),
        ],
        out_specs=pl.BlockSpec((B, N3), lambda k: (0, 0)),
        scratch_shapes=[pltpu.VMEM((B, N1), jnp.float32)],
        compiler_params=pltpu.CompilerParams(
            dimension_semantics=("arbitrary",)),
    )(x, w1, b1, w2, b2, w3, b3)


def kernel(x_nchw, block0_w, block0_b, block0_s, block0_t,
           block1_w, block1_b, block1_s, block1_t,
           block2_w, block2_b, block2_s, block2_t,
           block3_w, block3_b, block3_s, block3_t,
           fc1_w, fc1_b, fc2_w, fc2_b, fc3_w, fc3_b):
    x = jnp.transpose(x_nchw, (0, 2, 3, 1)).astype(jnp.bfloat16)
    x = _conv_block(x, block0_w, block0_b, block0_s, block0_t, 3, 32)
    x = _conv_block(x, block1_w, block1_b, block1_s, block1_t, 32, 64)
    x = _conv_block(x, block2_w, block2_b, block2_s, block2_t, 64, 128)
    x = _conv_block(x, block3_w, block3_b, block3_s, block3_t, 128, 256)
    B = x.shape[0]
    x = jnp.transpose(x, (0, 3, 1, 2)).reshape(B, -1)
    return _fc_head(x, fc1_w, fc1_b, fc2_w, fc2_b, fc3_w, fc3_b)


# speedup vs baseline: 16.9030x; 2.1379x over previous
import functools

import jax
import jax.numpy as jnp
from jax.experimental import pallas as pl
from jax.experimental.pallas import tpu as pltpu


def _cv_kernel(x_ref, w_ref, b_ref, s_ref, t_ref, o_ref, *,
               H, NG, C, U, OC, PH, PW):
    T = U + 2
    M = (H - 2) * NG
    R = H * NG
    nr = -(-T // U)

    pieces = []
    for ki in range(3):
        for r in range(nr):
            wpx = min(U, T - r * U)
            o = ki * NG + r
            L = min(M, R - o)
            sl = x_ref[0, o:o + L, 0:wpx * C]
            if L < M:
                sl = jnp.concatenate(
                    [sl, jnp.zeros((M - L, wpx * C), sl.dtype)], axis=0)
            pieces.append(sl)
    X4 = jnp.concatenate(pieces, axis=-1)
    z = jnp.dot(X4, w_ref[...], preferred_element_type=jnp.float32)
    z = z + b_ref[...]
    z = jnp.maximum(z, 0.0)
    z = z * s_ref[...] + t_ref[...]

    if U > 1:
        zw = jnp.concatenate(
            [jnp.maximum(z[:, (2 * p) * OC:(2 * p + 1) * OC],
                         z[:, (2 * p + 1) * OC:(2 * p + 2) * OC])
             for p in range(U // 2)], axis=-1).astype(o_ref.dtype)
        zm = jnp.maximum(zw[:M - NG], zw[NG:])
        for ph in range(PH):
            o_ref[0, ph * NG:(ph + 1) * NG, :] = \
                zm[2 * ph * NG:2 * ph * NG + NG]
    else:
        zh = jnp.maximum(z[:M - NG], z[NG:])
        for ph in range(PH):
            base = 2 * ph * NG
            rows = jnp.concatenate(
                [jnp.maximum(zh[base + 2 * p:base + 2 * p + 1],
                             zh[base + 2 * p + 1:base + 2 * p + 2])
                 for p in range(PW)], axis=0)
            o_ref[0, ph] = rows.astype(o_ref.dtype)


def _expand_w(bw, bb, bs, bt, C, OC, U):
    T = U + 2
    w9 = bw.reshape(3, 3, C, OC)
    cols = []
    for u in range(U):
        zl = jnp.zeros((3, u, C, OC), bw.dtype)
        zr = jnp.zeros((3, T - 3 - u, C, OC), bw.dtype)
        cols.append(jnp.concatenate([zl, w9, zr], axis=1))
    wt = jnp.stack(cols, axis=3)
    wt = wt.reshape(3 * T * C, U * OC)
    tile = lambda v: jnp.concatenate([v] * U, axis=1)
    return wt, tile(bb), tile(bs), tile(bt)


def _conv_stage(x, bw, bb, bs, bt, *, H, NG, C, U, OC, PW=None):
    B = x.shape[0]
    PH = (H - 2) // 2
    wt, b2, s2, t2 = _expand_w(bw, bb, bs, bt, C, OC, U)
    T = U + 2
    if U > 1:
        out_shape = jax.ShapeDtypeStruct((B, PH * NG, (U // 2) * OC),
                                         jnp.bfloat16)
        out_spec = pl.BlockSpec((1, PH * NG, (U // 2) * OC),
                                lambda b: (b, 0, 0))
    else:
        out_shape = jax.ShapeDtypeStruct((B, PH, PW, OC), jnp.bfloat16)
        out_spec = pl.BlockSpec((1, PH, PW, OC), lambda b: (b, 0, 0, 0))
    vec = pl.BlockSpec((1, U * OC), lambda b: (0, 0))
    kern = functools.partial(_cv_kernel, H=H, NG=NG, C=C, U=U, OC=OC,
                             PH=PH, PW=PW)
    return pl.pallas_call(
        kern,
        out_shape=out_shape,
        grid=(B,),
        in_specs=[
            pl.BlockSpec((1, H * NG, U * C), lambda b: (b, 0, 0)),
            pl.BlockSpec((3 * T * C, U * OC), lambda b: (0, 0)),
            vec, vec, vec,
        ],
        out_specs=out_spec,
        compiler_params=pltpu.CompilerParams(
            dimension_semantics=("parallel",)),
    )(x, wt, b2, s2, t2)


def _fc_kernel(x_ref, w1_ref, b1_ref, w2_ref, b2_ref, w3_ref, b3_ref,
               o_ref, acc_ref, *, nk):
    k = pl.program_id(0)

    @pl.when(k == 0)
    def _():
        acc_ref[...] = jnp.zeros_like(acc_ref)

    acc_ref[...] += jnp.dot(x_ref[...], w1_ref[...],
                            preferred_element_type=jnp.float32)

    @pl.when(k == nk - 1)
    def _():
        h1 = jnp.maximum(acc_ref[...] + b1_ref[...], 0.0).astype(jnp.bfloat16)
        z2 = jnp.dot(h1, w2_ref[...], preferred_element_type=jnp.float32)
        h2 = jnp.maximum(z2 + b2_ref[...], 0.0).astype(jnp.bfloat16)
        z3 = jnp.dot(h2, w3_ref[...], preferred_element_type=jnp.float32)
        o_ref[...] = z3 + b3_ref[...]


def _fc_head(x, w1, b1, w2, b2, w3, b3, tk=4608):
    B, K = x.shape
    nk = K // tk
    N1, N2, N3 = w1.shape[1], w2.shape[1], w3.shape[1]
    return pl.pallas_call(
        functools.partial(_fc_kernel, nk=nk),
        out_shape=jax.ShapeDtypeStruct((B, N3), jnp.float32),
        grid=(nk,),
        in_specs=[
            pl.BlockSpec((B, tk), lambda k: (0, k)),
            pl.BlockSpec((tk, N1), lambda k: (k, 0)),
            pl.BlockSpec((1, N1), lambda k: (0, 0)),
            pl.BlockSpec((N1, N2), lambda k: (0, 0)),
            pl.BlockSpec((1, N2), lambda k: (0, 0)),
            pl.BlockSpec((N2, N3), lambda k: (0, 0)),
            pl.BlockSpec((1, N3), lambda k: (0, 0)),
        ],
        out_specs=pl.BlockSpec((B, N3), lambda k: (0, 0)),
        scratch_shapes=[pltpu.VMEM((B, N1), jnp.float32)],
        compiler_params=pltpu.CompilerParams(
            dimension_semantics=("arbitrary",)),
    )(x, w1, b1, w2, b2, w3, b3)


def kernel(x_nchw, block0_w, block0_b, block0_s, block0_t,
           block1_w, block1_b, block1_s, block1_t,
           block2_w, block2_b, block2_s, block2_t,
           block3_w, block3_b, block3_s, block3_t,
           fc1_w, fc1_b, fc2_w, fc2_b, fc3_w, fc3_b):
    B, Cin, H0, W0 = x_nchw.shape
    U0 = 8
    NG = -(-W0 // (2 * U0)) * 2
    Wp = NG * U0
    x = jnp.pad(x_nchw, ((0, 0), (0, 0), (0, 0), (0, Wp - W0)))
    x = x.reshape(B, Cin, H0, NG, U0).transpose(0, 2, 3, 4, 1)
    x = x.reshape(B, H0 * NG, U0 * Cin).astype(jnp.bfloat16)

    x = _conv_stage(x, block0_w, block0_b, block0_s, block0_t,
                    H=H0, NG=NG, C=Cin, U=8, OC=32)
    x = _conv_stage(x, block1_w, block1_b, block1_s, block1_t,
                    H=110, NG=NG, C=32, U=4, OC=64)
    x = _conv_stage(x, block2_w, block2_b, block2_s, block2_t,
                    H=54, NG=NG, C=64, U=2, OC=128)
    x = _conv_stage(x, block3_w, block3_b, block3_s, block3_t,
                    H=26, NG=NG, C=128, U=1, OC=256, PW=12)
    x = jnp.transpose(x, (0, 3, 1, 2)).reshape(B, -1)
    return _fc_head(x, fc1_w, fc1_b, fc2_w, fc2_b, fc3_w, fc3_b)
